# Initial kernel scaffold; baseline (speedup 1.0000x reference)
import jax
import jax.numpy as jnp
from jax.experimental import pallas as pl


def kernel(x, edge_index, W1, b1, W2, b2, W3, b3):
    raise NotImplementedError("write your pallas kernel here")


# speedup vs baseline: 4.5703x; 4.5703x over previous
import functools

import jax
import jax.numpy as jnp
from jax import lax
from jax.experimental import pallas as pl
from jax.experimental.pallas import tpu as pltpu
from jax.experimental.pallas import tpu_sc as plsc

_N = 10000
_D = 128
_E = 320000
_CH = 128
_NCHUNK = 2560
_EPAD = _NCHUNK * _CH - _E
_NW = 32
_CPW = _NCHUNK // _NW
_NPAD = 10240
_TRASH = _N

_mesh = plsc.VectorSubcoreMesh(core_axis_name="c", subcore_axis_name="s")


_HPT = _NCHUNK // 16

@functools.partial(
    pl.kernel,
    out_type=jax.ShapeDtypeStruct((_NPAD,), jnp.float32),
    mesh=_mesh,
    scratch_types=[
        pltpu.VMEM((_HPT, _CH), jnp.int32),
        pltpu.VMEM((_HPT, _CH), jnp.int32),
        pltpu.VMEM((_CH,), jnp.float32),
        pltpu.VMEM((_NPAD // 32,), jnp.float32),
        pltpu.VMEM((_NPAD // 32,), jnp.float32),
        pltpu.VMEM((_NPAD // 32,), jnp.float32),
        pltpu.VMEM_SHARED((_NPAD,), jnp.float32),
        pltpu.VMEM_SHARED((_NPAD,), jnp.float32),
    ],
)
def _hist_kernel(src2d, dst2d, zeros_hist, ones_row, w_out,
                 src_v, dst_v, ones_v, cbuf, dbuf, wbuf, cnt_sh, deg_sh):
    c = lax.axis_index("c")
    t = lax.axis_index("s")
    zoff = pl.multiple_of(t * (_NPAD // 16), 8)
    pltpu.sync_copy(zeros_hist, cnt_sh.at[pl.ds(zoff, _NPAD // 16)])
    pltpu.sync_copy(zeros_hist, deg_sh.at[pl.ds(zoff, _NPAD // 16)])
    pltpu.sync_copy(ones_row, ones_v)

    cs = pl.multiple_of(t * _HPT, 8)
    pltpu.sync_copy(src2d.at[pl.ds(cs, _HPT)], src_v)
    pltpu.sync_copy(dst2d.at[pl.ds(cs, _HPT)], dst_v)
    plsc.subcore_barrier()

    def body(i, carry):
        pltpu.sync_copy(ones_v, deg_sh.at[src_v.at[i]], add=True)
        pltpu.sync_copy(ones_v, cnt_sh.at[dst_v.at[i]], add=True)
        return carry

    lax.fori_loop(0, _HPT, body, 0)
    plsc.subcore_barrier()

    nb = _NPAD // 32
    base = pl.multiple_of(c * (_NPAD // 2) + t * nb, 8)
    pltpu.sync_copy(cnt_sh.at[pl.ds(base, nb)], cbuf)
    pltpu.sync_copy(deg_sh.at[pl.ds(base, nb)], dbuf)

    def wbody(i, carry):
        cnt = cbuf[pl.ds(i * 16, 16)] + 1.0
        deg = dbuf[pl.ds(i * 16, 16)] + 1.0
        wbuf[pl.ds(i * 16, 16)] = 1.0 / (cnt * deg)
        return carry

    lax.fori_loop(0, nb // 16, wbody, 0)
    pltpu.sync_copy(wbuf, w_out.at[pl.ds(base, nb)])


_RPT = _NPAD // 16

@functools.partial(
    pl.kernel,
    out_type=(
        jax.ShapeDtypeStruct((_N, _D), jnp.float32),
        jax.ShapeDtypeStruct((_N, _D), jnp.float32),
    ),
    mesh=_mesh,
    scratch_types=[
        pltpu.VMEM((_CPW, _CH), jnp.int32),
        pltpu.VMEM((_CPW, _CH), jnp.int32),
        pltpu.VMEM((_CH, _D), jnp.float32),
        pltpu.VMEM_SHARED((_NPAD, _D), jnp.float32),
        pltpu.SemaphoreType.DMA,
    ],
)
def _agg_kernel(z_hbm, src2d, dst2d, zeros_rows, out0, out1,
                src_v, dst_v, rows_v, acc_sh, gsem):
    c = lax.axis_index("c")
    t = lax.axis_index("s")
    w = t * 2 + c

    zoff = pl.multiple_of(t * _RPT, 8)
    pltpu.sync_copy(zeros_rows, acc_sh.at[pl.ds(zoff, _RPT)])

    cs = pl.multiple_of(w * _CPW, 8)
    pltpu.sync_copy(src2d.at[pl.ds(cs, _CPW)], src_v)
    pltpu.sync_copy(dst2d.at[pl.ds(cs, _CPW)], dst_v)
    plsc.subcore_barrier()

    def body(i, carry):
        pltpu.async_copy(z_hbm.at[src_v.at[i]], rows_v, gsem).wait()
        pltpu.sync_copy(rows_v, acc_sh.at[dst_v.at[i]], add=True)
        return carry

    lax.fori_loop(0, _CPW, body, 0)
    plsc.subcore_barrier()

    nrows = jnp.minimum(_RPT, _N - t * _RPT)

    @pl.when(c == 0)
    def _():
        @pl.when(nrows == _RPT)
        def _():
            pltpu.sync_copy(acc_sh.at[pl.ds(zoff, _RPT)],
                            out0.at[pl.ds(zoff, _RPT)])

        @pl.when(nrows < _RPT)
        def _():
            pltpu.sync_copy(acc_sh.at[pl.ds(zoff, _N - 15 * _RPT)],
                            out0.at[pl.ds(zoff, _N - 15 * _RPT)])

    @pl.when(c == 1)
    def _():
        @pl.when(nrows == _RPT)
        def _():
            pltpu.sync_copy(acc_sh.at[pl.ds(zoff, _RPT)],
                            out1.at[pl.ds(zoff, _RPT)])

        @pl.when(nrows < _RPT)
        def _():
            pltpu.sync_copy(acc_sh.at[pl.ds(zoff, _N - 15 * _RPT)],
                            out1.at[pl.ds(zoff, _N - 15 * _RPT)])


_RB = 1000


def _l1_body(x_ref, W_ref, b_ref, o_ref):
    acc = jnp.dot(x_ref[...], W_ref[...], preferred_element_type=jnp.float32)
    o_ref[...] = jnp.maximum(acc + b_ref[...], 0.0)


def _layer1(x, W, b):
    return pl.pallas_call(
        _l1_body,
        grid=(_N // _RB,),
        in_specs=[
            pl.BlockSpec((_RB, _D), lambda i: (i, 0)),
            pl.BlockSpec((_D, _D), lambda i: (0, 0)),
            pl.BlockSpec((1, _D), lambda i: (0, 0)),
        ],
        out_specs=pl.BlockSpec((_RB, _D), lambda i: (i, 0)),
        out_shape=jax.ShapeDtypeStruct((_N, _D), jnp.float32),
    )(x, W, b.reshape(1, _D))


def _mid_body(z_ref, s0_ref, s1_ref, w_ref, W_ref, b_ref, o_ref):
    z = z_ref[...]
    zin = z - (s0_ref[...] + s1_ref[...] + z) * w_ref[...]
    acc = jnp.dot(zin, W_ref[...], preferred_element_type=jnp.float32)
    o_ref[...] = jnp.maximum(acc + b_ref[...], 0.0)


def _layer_mid(z, s0, s1, w, W, b):
    return pl.pallas_call(
        _mid_body,
        grid=(_N // _RB,),
        in_specs=[
            pl.BlockSpec((_RB, _D), lambda i: (i, 0)),
            pl.BlockSpec((_RB, _D), lambda i: (i, 0)),
            pl.BlockSpec((_RB, _D), lambda i: (i, 0)),
            pl.BlockSpec((_RB, 1), lambda i: (i, 0)),
            pl.BlockSpec((_D, _D), lambda i: (0, 0)),
            pl.BlockSpec((1, _D), lambda i: (0, 0)),
        ],
        out_specs=pl.BlockSpec((_RB, _D), lambda i: (i, 0)),
        out_shape=jax.ShapeDtypeStruct((_N, _D), jnp.float32),
    )(z, s0, s1, w, W, b.reshape(1, _D))


def kernel(x, edge_index, W1, b1, W2, b2, W3, b3):
    src = edge_index[0].astype(jnp.int32)
    dst = edge_index[1].astype(jnp.int32)
    trash = jnp.full((_EPAD,), _TRASH, jnp.int32)
    src_h = jnp.concatenate([src, trash]).reshape(_NCHUNK, _CH)
    dst_h = jnp.concatenate([dst, trash]).reshape(_NCHUNK, _CH)
    src_a = jnp.concatenate([src, jnp.zeros((_EPAD,), jnp.int32)]
                            ).reshape(_NCHUNK, _CH)

    zeros_hist = jnp.zeros((_NPAD // 16,), jnp.float32)
    ones_row = jnp.ones((_CH,), jnp.float32)
    zeros_rows = jnp.zeros((_RPT, _D), jnp.float32)

    w_pad = _hist_kernel(src_h, dst_h, zeros_hist, ones_row)
    w = w_pad[:_N].reshape(_N, 1)

    z1 = _layer1(x, W1, b1)
    s0, s1 = _agg_kernel(z1, src_a, dst_h, zeros_rows)
    z2 = _layer_mid(z1, s0, s1, w, W2, b2)
    s0, s1 = _agg_kernel(z2, src_a, dst_h, zeros_rows)
    z3 = _layer_mid(z2, s0, s1, w, W3, b3)
    return z3
